# baseline (speedup 1.0000x reference)
import jax
import jax.numpy as jnp
from jax import lax
from jax.experimental import pallas as pl
from jax.experimental.pallas import tpu as pltpu
from jax.experimental.pallas import tpu_sc as plsc

_B = 16384
_D = 16
_L = 16
_V = 2_000_000

_info = plsc.get_sparse_core_info()
_NC, _NS = _info.num_cores, _info.num_subcores
_NW = _NC * _NS
_BPW = _B // _NW


def _fm_body(gidx_hbm, qflat_hbm, lin_hbm, w0_hbm, out_hbm,
             uv_v, iv_v, pb_v, pidx_v, gat_v, lg_v, out_v, w0_v, sem, lsem):
    wid = lax.axis_index("s") * _NC + lax.axis_index("c")
    base = wid * _BPW

    pltpu.sync_copy(gidx_hbm.at[pl.ds(base, _BPW)], uv_v)
    pltpu.sync_copy(gidx_hbm.at[pl.ds(_B + base, _BPW)], iv_v)
    pltpu.sync_copy(w0_hbm, w0_v)

    for c in range(_BPW // 128):
        pltpu.async_copy(lin_hbm.at[uv_v.at[pl.ds(c * 128, 128)]],
                         lg_v.at[pl.ds(c * 128, 128)], lsem)
        pltpu.async_copy(lin_hbm.at[iv_v.at[pl.ds(c * 128, 128)]],
                         lg_v.at[pl.ds(_BPW + c * 128, 128)], lsem)

    def pb_u(k, carry):
        v = uv_v[pl.ds(k * _L, _L)]
        pb_v[pl.ds(k * _L, _L)] = ((v >> 7) << 10) + (v & 127)
        return carry

    def pb_i(k, carry):
        v = iv_v[pl.ds(k * _L, _L)]
        pb_v[pl.ds(_BPW + k * _L, _L)] = ((v >> 7) << 10) + (v & 127)
        return carry

    lax.fori_loop(0, _BPW // _L, pb_u, 0)
    lax.fori_loop(0, _BPW // _L, pb_i, 0)

    for d in range(_D):
        cd = (d // 8) * (_V * 8) + (d % 8) * 128

        def pidx_block(k, carry, d=d, cd=cd):
            pidx_v[d, pl.ds(k * _L, _L)] = pb_v[pl.ds(k * _L, _L)] + cd
            return carry

        lax.fori_loop(0, 2 * _BPW // _L, pidx_block, 0)
        for c in range(2 * _BPW // 128):
            pltpu.async_copy(
                qflat_hbm.at[pidx_v.at[d, pl.ds(c * 128, 128)]],
                gat_v.at[pl.ds(d * 2 * _BPW + c * 128, 128)], sem)

    pltpu.make_async_copy(gidx_hbm.at[pl.ds(0, 2 * _BPW * _D)], gat_v,
                          sem).wait()
    pltpu.make_async_copy(lin_hbm.at[pl.ds(0, 2 * _BPW)], lg_v, lsem).wait()

    w0 = w0_v[...]

    def block(i, carry):
        o = i * _L
        acc = w0 + lg_v[pl.ds(o, _L)] + lg_v[pl.ds(_BPW + o, _L)]
        for d in range(_D):
            u = gat_v[pl.ds(d * 2 * _BPW + o, _L)]
            t = gat_v[pl.ds(d * 2 * _BPW + _BPW + o, _L)]
            acc = acc + u * t
        out_v[pl.ds(o, _L)] = acc
        return carry

    lax.fori_loop(0, _BPW // _L, block, 0)

    pltpu.sync_copy(out_v, out_hbm.at[pl.ds(base, _BPW)])


@jax.jit
def _fm(gidx, qflat, lin_flat, w0_16):
    run = pl.kernel(
        _fm_body,
        out_type=jax.ShapeDtypeStruct((_B,), jnp.float32),
        mesh=plsc.VectorSubcoreMesh(core_axis_name="c", subcore_axis_name="s"),
        scratch_types=[
            pltpu.VMEM((_BPW,), jnp.int32),
            pltpu.VMEM((_BPW,), jnp.int32),
            pltpu.VMEM((2 * _BPW,), jnp.int32),
            pltpu.VMEM((_D, 2 * _BPW), jnp.int32),
            pltpu.VMEM((_D * 2 * _BPW,), jnp.float32),
            pltpu.VMEM((2 * _BPW,), jnp.float32),
            pltpu.VMEM((_BPW,), jnp.float32),
            pltpu.VMEM((_L,), jnp.float32),
            pltpu.SemaphoreType.DMA,
            pltpu.SemaphoreType.DMA,
        ],
        compiler_params=pltpu.CompilerParams(
            needs_layout_passes=False, use_tc_tiling_on_sc=False,
            skip_device_barrier=True, disable_bounds_checks=True,
            disable_semaphore_checks=True),
    )
    return run(gidx, qflat, lin_flat, w0_16)


def kernel(user_item, offset, linear_emb, quad_emb, w_0):
    gidx = (user_item + offset[None, :]).T.reshape(-1)
    qflat = (quad_emb.T.reshape(2, 8, _V // 128, 128)
             .transpose(0, 2, 1, 3).reshape(-1))
    lin_flat = lax.bitcast_convert_type(
        lax.bitcast_convert_type(linear_emb, jnp.uint8).reshape(_V, 4),
        jnp.float32)
    w0_16 = jnp.broadcast_to(w_0, (_L,)).astype(jnp.float32)
    return _fm(gidx, qflat, lin_flat, w0_16)


# speedup vs baseline: 1.3415x; 1.3415x over previous
import jax
import jax.numpy as jnp
from jax import lax
from jax.experimental import pallas as pl
from jax.experimental.pallas import tpu as pltpu
from jax.experimental.pallas import tpu_sc as plsc

_B = 16384
_D = 16
_L = 16
_V = 2_000_000

_info = plsc.get_sparse_core_info()
_NC, _NS = _info.num_cores, _info.num_subcores
_NW = _NC * _NS
_BPW = _B // _NW


def _fm_body(gidx_hbm, qflat_hbm, w0_hbm, out_hbm,
             uv_v, iv_v, pb_v, pidx_v, gat_v, out_v, w0_v, sem):
    wid = lax.axis_index("s") * _NC + lax.axis_index("c")
    base = wid * _BPW

    pltpu.sync_copy(gidx_hbm.at[pl.ds(base, _BPW)], uv_v)
    pltpu.sync_copy(gidx_hbm.at[pl.ds(_B + base, _BPW)], iv_v)
    pltpu.sync_copy(w0_hbm, w0_v)

    def pb_u(k, carry):
        v = uv_v[pl.ds(k * _L, _L)]
        pb_v[pl.ds(k * _L, _L)] = ((v >> 7) << 10) + (v & 127)
        return carry

    def pb_i(k, carry):
        v = iv_v[pl.ds(k * _L, _L)]
        pb_v[pl.ds(_BPW + k * _L, _L)] = ((v >> 7) << 10) + (v & 127)
        return carry

    lax.fori_loop(0, _BPW // _L, pb_u, 0)
    lax.fori_loop(0, _BPW // _L, pb_i, 0)

    for d in range(_D):
        cd = (d // 8) * (_V * 8) + (d % 8) * 128

        def pidx_block(k, carry, d=d, cd=cd):
            pidx_v[d, pl.ds(k * _L, _L)] = pb_v[pl.ds(k * _L, _L)] + cd
            return carry

        lax.fori_loop(0, 2 * _BPW // _L, pidx_block, 0)
        for c in range(2 * _BPW // 128):
            pltpu.async_copy(
                qflat_hbm.at[pidx_v.at[d, pl.ds(c * 128, 128)]],
                gat_v.at[pl.ds(d * 2 * _BPW + c * 128, 128)], sem)

    pltpu.make_async_copy(gidx_hbm.at[pl.ds(0, 2 * _BPW * _D)], gat_v,
                          sem).wait()

    w0 = w0_v[...]

    def block(i, carry):
        o = i * _L
        acc = w0
        for d in range(_D):
            u = gat_v[pl.ds(d * 2 * _BPW + o, _L)]
            t = gat_v[pl.ds(d * 2 * _BPW + _BPW + o, _L)]
            acc = acc + u * t
        out_v[pl.ds(o, _L)] = acc
        return carry

    lax.fori_loop(0, _BPW // _L, block, 0)

    pltpu.sync_copy(out_v, out_hbm.at[pl.ds(base, _BPW)])


@jax.jit
def _fm(gidx, qflat, w0_16):
    run = pl.kernel(
        _fm_body,
        out_type=jax.ShapeDtypeStruct((_B,), jnp.float32),
        mesh=plsc.VectorSubcoreMesh(core_axis_name="c", subcore_axis_name="s"),
        scratch_types=[
            pltpu.VMEM((_BPW,), jnp.int32),
            pltpu.VMEM((_BPW,), jnp.int32),
            pltpu.VMEM((2 * _BPW,), jnp.int32),
            pltpu.VMEM((_D, 2 * _BPW), jnp.int32),
            pltpu.VMEM((_D * 2 * _BPW,), jnp.float32),
            pltpu.VMEM((_BPW,), jnp.float32),
            pltpu.VMEM((_L,), jnp.float32),
            pltpu.SemaphoreType.DMA,
        ],
        compiler_params=pltpu.CompilerParams(
            needs_layout_passes=False, use_tc_tiling_on_sc=False,
            skip_device_barrier=True, disable_bounds_checks=True,
            disable_semaphore_checks=True),
    )
    return run(gidx, qflat, w0_16)


def kernel(user_item, offset, linear_emb, quad_emb, w_0):
    idx2 = user_item + offset[None, :]
    gidx = idx2.T.reshape(-1)
    qflat = (quad_emb.T.reshape(2, 8, _V // 128, 128)
             .transpose(0, 2, 1, 3).reshape(-1))
    w0_16 = jnp.broadcast_to(w_0, (_L,)).astype(jnp.float32)
    lin_tot = jnp.take(linear_emb, idx2, axis=0).sum(axis=(1, 2))
    return _fm(gidx, qflat, w0_16) + lin_tot


# speedup vs baseline: 1.7801x; 1.3270x over previous
import jax
import jax.numpy as jnp
from jax import lax
from jax.experimental import pallas as pl
from jax.experimental.pallas import tpu as pltpu
from jax.experimental.pallas import tpu_sc as plsc

_B = 16384
_D = 16
_L = 16
_V = 2_000_000

_info = plsc.get_sparse_core_info()
_NC, _NS = _info.num_cores, _info.num_subcores
_NW = _NC * _NS
_BPW = _B // _NW


def _fm_body(gidx_hbm, qflat_hbm, lin_hbm, w0_hbm, out_hbm,
             uv_v, iv_v, pb_v, pidx_v, gat_v, lg_v, out_v, w0_v, sem, lsem):
    wid = lax.axis_index("s") * _NC + lax.axis_index("c")
    base = wid * _BPW

    pltpu.sync_copy(gidx_hbm.at[pl.ds(base, _BPW)], uv_v)
    pltpu.sync_copy(gidx_hbm.at[pl.ds(_B + base, _BPW)], iv_v)
    pltpu.sync_copy(w0_hbm, w0_v)

    for c in range(_BPW // 128):
        pltpu.async_copy(lin_hbm.at[uv_v.at[pl.ds(c * 128, 128)]],
                         lg_v.at[pl.ds(c * 128, 128)], lsem)
        pltpu.async_copy(lin_hbm.at[iv_v.at[pl.ds(c * 128, 128)]],
                         lg_v.at[pl.ds(_BPW + c * 128, 128)], lsem)

    def pb_u(k, carry):
        v = uv_v[pl.ds(k * _L, _L)]
        pb_v[pl.ds(k * _L, _L)] = ((v >> 7) << 10) + (v & 127)
        return carry

    def pb_i(k, carry):
        v = iv_v[pl.ds(k * _L, _L)]
        pb_v[pl.ds(_BPW + k * _L, _L)] = ((v >> 7) << 10) + (v & 127)
        return carry

    lax.fori_loop(0, _BPW // _L, pb_u, 0)
    lax.fori_loop(0, _BPW // _L, pb_i, 0)

    for d in range(_D):
        cd = (d // 8) * (_V * 8) + (d % 8) * 128

        def pidx_block(k, carry, d=d, cd=cd):
            pidx_v[d, pl.ds(k * _L, _L)] = pb_v[pl.ds(k * _L, _L)] + cd
            return carry

        lax.fori_loop(0, 2 * _BPW // _L, pidx_block, 0)
        for c in range(2 * _BPW // 128):
            pltpu.async_copy(
                qflat_hbm.at[pidx_v.at[d, pl.ds(c * 128, 128)]],
                gat_v.at[pl.ds(d * 2 * _BPW + c * 128, 128)], sem)

    pltpu.make_async_copy(gidx_hbm.at[pl.ds(0, 2 * _BPW * _D)], gat_v,
                          sem).wait()
    pltpu.make_async_copy(lin_hbm.at[pl.ds(0, 2 * _BPW)], lg_v, lsem).wait()

    w0 = w0_v[...]

    def block(i, carry):
        o = i * _L
        acc = w0 + lg_v[pl.ds(o, _L)] + lg_v[pl.ds(_BPW + o, _L)]
        for d in range(_D):
            u = gat_v[pl.ds(d * 2 * _BPW + o, _L)]
            t = gat_v[pl.ds(d * 2 * _BPW + _BPW + o, _L)]
            acc = acc + u * t
        out_v[pl.ds(o, _L)] = acc
        return carry

    lax.fori_loop(0, _BPW // _L, block, 0)

    pltpu.sync_copy(out_v, out_hbm.at[pl.ds(base, _BPW)])


@jax.jit
def _fm(gidx, qflat, lin_flat, w0_16):
    run = pl.kernel(
        _fm_body,
        out_type=jax.ShapeDtypeStruct((_B,), jnp.float32),
        mesh=plsc.VectorSubcoreMesh(core_axis_name="c", subcore_axis_name="s"),
        scratch_types=[
            pltpu.VMEM((_BPW,), jnp.int32),
            pltpu.VMEM((_BPW,), jnp.int32),
            pltpu.VMEM((2 * _BPW,), jnp.int32),
            pltpu.VMEM((_D, 2 * _BPW), jnp.int32),
            pltpu.VMEM((_D * 2 * _BPW,), jnp.float32),
            pltpu.VMEM((2 * _BPW,), jnp.float32),
            pltpu.VMEM((_BPW,), jnp.float32),
            pltpu.VMEM((_L,), jnp.float32),
            pltpu.SemaphoreType.DMA,
            pltpu.SemaphoreType.DMA,
        ],
        compiler_params=pltpu.CompilerParams(
            needs_layout_passes=False, use_tc_tiling_on_sc=False,
            skip_device_barrier=True, disable_bounds_checks=True,
            disable_semaphore_checks=True),
    )
    return run(gidx, qflat, lin_flat, w0_16)


def kernel(user_item, offset, linear_emb, quad_emb, w_0):
    gidx = (user_item + offset[None, :]).T.reshape(-1)
    qflat = (quad_emb.T.reshape(2, 8, _V // 128, 128)
             .transpose(0, 2, 1, 3).reshape(-1))
    lin_flat = linear_emb.reshape(-1)
    w0_16 = jnp.broadcast_to(w_0, (_L,)).astype(jnp.float32)
    return _fm(gidx, qflat, lin_flat, w0_16)
